# baseline (speedup 1.0000x reference)
import jax
import jax.numpy as jnp
from jax import lax
from jax.experimental import pallas as pl
from jax.experimental.pallas import tpu as pltpu
from jax.experimental.pallas import tpu_sc as plsc

N = 262144
D = 64
C = 85
L = 16
NC = 2
NS = 16
NW = NC * NS
ROWS_PER_W = N // NW
CHUNK = 128
N_CHUNKS = ROWS_PER_W // CHUNK
NBUF = 4
UNROLL = 4


def _sc_body(labels_hbm, features_hbm, centers_hbm, out_hbm, *scr):
    lbufs = scr[0:NBUF]
    fbufs = scr[NBUF:2 * NBUF]
    cbufs = scr[2 * NBUF:3 * NBUF]
    acc_v = scr[3 * NBUF]
    lsems = scr[3 * NBUF + 1:3 * NBUF + 1 + NBUF]
    gsems = scr[3 * NBUF + 1 + NBUF:3 * NBUF + 1 + 2 * NBUF]
    fsems = scr[3 * NBUF + 1 + 2 * NBUF:3 * NBUF + 1 + 3 * NBUF]

    wid = lax.axis_index("s") * NC + lax.axis_index("c")
    base = wid * ROWS_PER_W

    def issue_label(ci, b):
        pltpu.async_copy(labels_hbm.at[pl.ds(base + ci * CHUNK, CHUNK)],
                         lbufs[b], lsems[b])

    def issue_gather(ci, b):
        del ci
        pltpu.async_copy(centers_hbm.at[lbufs[b]], cbufs[b], gsems[b])

    def issue_feat(ci, b):
        pltpu.async_copy(features_hbm.at[pl.ds(base + ci * CHUNK, CHUNK), :],
                         fbufs[b], fsems[b])

    def wait_label(b):
        pltpu.make_async_copy(labels_hbm.at[pl.ds(base, CHUNK)],
                              lbufs[b], lsems[b]).wait()

    def wait_gather(b):
        pltpu.make_async_copy(centers_hbm.at[lbufs[b]],
                              cbufs[b], gsems[b]).wait()

    def wait_feat(b):
        pltpu.make_async_copy(features_hbm.at[pl.ds(base, CHUNK), :],
                              fbufs[b], fsems[b]).wait()

    for j in range(NBUF):
        issue_label(j, j)
    for j in range(2):
        wait_label(j)
        issue_gather(j, j)
        issue_feat(j, j)

    def compute_chunk(fb, cb, accs):
        def row_body(t, accs):
            a0, a1, a2, a3 = accs
            i0 = t * UNROLL
            for r in range(UNROLL):
                i = i0 + r
                f0 = fb[i, pl.ds(0, L)]
                c0 = cb[i, pl.ds(0, L)]
                d0 = f0 - c0
                a0 = a0 + d0 * d0
                f1 = fb[i, pl.ds(L, L)]
                c1 = cb[i, pl.ds(L, L)]
                d1 = f1 - c1
                a1 = a1 + d1 * d1
                f2 = fb[i, pl.ds(2 * L, L)]
                c2 = cb[i, pl.ds(2 * L, L)]
                d2 = f2 - c2
                a2 = a2 + d2 * d2
                f3 = fb[i, pl.ds(3 * L, L)]
                c3 = cb[i, pl.ds(3 * L, L)]
                d3 = f3 - c3
                a3 = a3 + d3 * d3
            return (a0, a1, a2, a3)

        return lax.fori_loop(0, CHUNK // UNROLL, row_body, accs)

    def outer_body(q, accs):
        for b in range(NBUF):
            ci = q * NBUF + b
            wait_gather(b)
            wait_feat(b)
            b2 = (b + 2) % NBUF

            @pl.when(ci + 2 < N_CHUNKS)
            def _():
                wait_label(b2)
                issue_gather(ci + 2, b2)
                issue_feat(ci + 2, b2)

            @pl.when(ci + NBUF < N_CHUNKS)
            def _():
                issue_label(ci + NBUF, b)

            accs = compute_chunk(fbufs[b], cbufs[b], accs)
        return accs

    zero = jnp.zeros((L,), jnp.float32)
    a0, a1, a2, a3 = lax.fori_loop(0, N_CHUNKS // NBUF, outer_body,
                                   (zero, zero, zero, zero))
    acc_v[...] = (a0 + a1) + (a2 + a3)
    pltpu.sync_copy(acc_v, out_hbm.at[wid])


@jax.jit
def _center_loss(labels, features, centers):
    mesh = plsc.VectorSubcoreMesh(core_axis_name="c", subcore_axis_name="s")
    scratch = (
        [pltpu.VMEM((CHUNK,), jnp.int32) for _ in range(NBUF)]
        + [pltpu.VMEM((CHUNK, D), jnp.float32) for _ in range(NBUF)]
        + [pltpu.VMEM((CHUNK, D), jnp.float32) for _ in range(NBUF)]
        + [pltpu.VMEM((L,), jnp.float32)]
        + [pltpu.SemaphoreType.DMA for _ in range(3 * NBUF)]
    )
    partials = pl.kernel(
        _sc_body,
        out_type=jax.ShapeDtypeStruct((NW, L), jnp.float32),
        mesh=mesh,
        scratch_types=scratch,
        compiler_params=pltpu.CompilerParams(use_tc_tiling_on_sc=False),
    )(labels, features, centers)
    return 0.5 * jnp.sum(partials)


def kernel(labels, features, preds, centers):
    del preds
    return _center_loss(labels, features, centers)
